# baseline (speedup 1.0000x reference)
import jax
import jax.numpy as jnp
from jax.experimental import pallas as pl
from jax.experimental.pallas import tpu as pltpu


def _jacobi_body(mi_ref, x0_ref, cr_ref, cl_ref, cd_ref, cu_ref, invd_ref,
                 b_ref, out_ref):
    invd = invd_ref[...]
    ncr = invd * cr_ref[...]
    ncl = invd * cl_ref[...]
    ncd = invd * cd_ref[...]
    ncu = invd * cu_ref[...]
    ib = invd * b_ref[...]

    g, n = x0_ref.shape[0], x0_ref.shape[1]
    zc = jnp.zeros((g, n, 1), dtype=jnp.float32)
    zr = jnp.zeros((g, 1, n), dtype=jnp.float32)

    def sweep(x):
        xl = jnp.concatenate([x[:, :, 1:], zc], axis=2)
        xr = jnp.concatenate([zc, x[:, :, :-1]], axis=2)
        xd = jnp.concatenate([x[:, 1:, :], zr], axis=1)
        xu = jnp.concatenate([zr, x[:, :-1, :]], axis=1)
        return ib - ((ncr * xl + ncl * xr) + (ncd * xd + ncu * xu))

    mi = mi_ref[0]
    x = jax.lax.fori_loop(0, mi // 5,
                          lambda _, x: sweep(sweep(sweep(sweep(sweep(x))))),
                          x0_ref[...])
    out_ref[...] = jax.lax.fori_loop(0, mi % 5, lambda _, x: sweep(x), x)


def kernel(u, M_vals, invD_vals, b, rows, cols, maxiter):
    del rows, cols
    B = u.shape[0]
    n = u.shape[-1]
    E = n * (n - 1)
    original_shape = u.shape

    seg = M_vals.reshape(B, 4, E)
    cr = jnp.pad(seg[:, 0].reshape(B, n, n - 1), ((0, 0), (0, 0), (0, 1)))
    cl = jnp.pad(seg[:, 1].reshape(B, n, n - 1), ((0, 0), (0, 0), (1, 0)))
    cd = jnp.pad(seg[:, 2].reshape(B, n - 1, n), ((0, 0), (0, 1), (0, 0)))
    cu = jnp.pad(seg[:, 3].reshape(B, n - 1, n), ((0, 0), (1, 0), (0, 0)))

    x0 = u.reshape(B, n, n)
    invd = invD_vals.reshape(B, n, n)
    bg = b.reshape(B, n, n)
    mi = jnp.asarray(maxiter, dtype=jnp.int32).reshape(1)

    gsz = 2 if B % 2 == 0 else 1
    spec = pl.BlockSpec((gsz, n, n), lambda i, mi_: (i, 0, 0))
    out = pl.pallas_call(
        _jacobi_body,
        grid_spec=pltpu.PrefetchScalarGridSpec(
            num_scalar_prefetch=1,
            grid=(B // gsz,),
            in_specs=[spec] * 7,
            out_specs=spec,
        ),
        out_shape=jax.ShapeDtypeStruct((B, n, n), jnp.float32),
        compiler_params=pltpu.CompilerParams(
            dimension_semantics=("parallel",),
        ),
    )(mi, x0, cr, cl, cd, cu, invd, bg)

    return jax.lax.stop_gradient(out.reshape(original_shape))


# speedup vs baseline: 1.0178x; 1.0178x over previous
import jax
import jax.numpy as jnp
from jax.experimental import pallas as pl
from jax.experimental.pallas import tpu as pltpu


def _jacobi_body(mi_ref, x0_ref, cr_ref, cl_ref, cd_ref, cu_ref, invd_ref,
                 b_ref, out_ref, ncr_s, ncl_s, ncd_s, ncu_s, ib_s):
    invd = invd_ref[0]
    ncr_s[...] = invd * cr_ref[0]
    ncl_s[...] = invd * cl_ref[0]
    ncd_s[...] = invd * cd_ref[0]
    ncu_s[...] = invd * cu_ref[0]
    ib_s[...] = invd * b_ref[0]

    n = x0_ref.shape[1]
    zc = jnp.zeros((n, 1), dtype=jnp.float32)
    zr = jnp.zeros((1, n), dtype=jnp.float32)

    def sweep(x):
        xl = jnp.concatenate([x[:, 1:], zc], axis=1)
        xr = jnp.concatenate([zc, x[:, :-1]], axis=1)
        xd = jnp.concatenate([x[1:, :], zr], axis=0)
        xu = jnp.concatenate([zr, x[:-1, :]], axis=0)
        return ib_s[...] - ((ncr_s[...] * xl + ncl_s[...] * xr)
                            + (ncd_s[...] * xd + ncu_s[...] * xu))

    mi = mi_ref[0]
    x = jax.lax.fori_loop(0, mi // 5,
                          lambda _, x: sweep(sweep(sweep(sweep(sweep(x))))),
                          x0_ref[0])
    out_ref[0] = jax.lax.fori_loop(0, mi % 5, lambda _, x: sweep(x), x)


def kernel(u, M_vals, invD_vals, b, rows, cols, maxiter):
    del rows, cols
    B = u.shape[0]
    n = u.shape[-1]
    E = n * (n - 1)
    original_shape = u.shape

    seg = M_vals.reshape(B, 4, E)
    cr = jnp.pad(seg[:, 0].reshape(B, n, n - 1), ((0, 0), (0, 0), (0, 1)))
    cl = jnp.pad(seg[:, 1].reshape(B, n, n - 1), ((0, 0), (0, 0), (1, 0)))
    cd = jnp.pad(seg[:, 2].reshape(B, n - 1, n), ((0, 0), (0, 1), (0, 0)))
    cu = jnp.pad(seg[:, 3].reshape(B, n - 1, n), ((0, 0), (1, 0), (0, 0)))

    x0 = u.reshape(B, n, n)
    invd = invD_vals.reshape(B, n, n)
    bg = b.reshape(B, n, n)
    mi = jnp.asarray(maxiter, dtype=jnp.int32).reshape(1)

    spec = pl.BlockSpec((1, n, n), lambda i, mi_: (i, 0, 0))
    out = pl.pallas_call(
        _jacobi_body,
        grid_spec=pltpu.PrefetchScalarGridSpec(
            num_scalar_prefetch=1,
            grid=(B,),
            in_specs=[spec] * 7,
            out_specs=spec,
            scratch_shapes=[pltpu.VMEM((n, n), jnp.float32)] * 5,
        ),
        out_shape=jax.ShapeDtypeStruct((B, n, n), jnp.float32),
        compiler_params=pltpu.CompilerParams(
            dimension_semantics=("parallel",),
        ),
    )(mi, x0, cr, cl, cd, cu, invd, bg)

    return jax.lax.stop_gradient(out.reshape(original_shape))


# speedup vs baseline: 1.0235x; 1.0055x over previous
import jax
import jax.numpy as jnp
from jax.experimental import pallas as pl
from jax.experimental.pallas import tpu as pltpu


def _jacobi_body(mi_ref, x0_ref, cr_ref, cl_ref, cd_ref, cu_ref, invd_ref,
                 b_ref, out_ref):
    invd = invd_ref[0]
    ncr = invd * cr_ref[0]
    ncl = invd * cl_ref[0]
    ncd = invd * cd_ref[0]
    ncu = invd * cu_ref[0]
    ib = invd * b_ref[0]

    n = x0_ref.shape[1]
    zc = jnp.zeros((n, 1), dtype=jnp.float32)
    zr = jnp.zeros((1, n), dtype=jnp.float32)

    def sweep(x):
        xl = jnp.concatenate([x[:, 1:], zc], axis=1)
        xr = jnp.concatenate([zc, x[:, :-1]], axis=1)
        xd = jnp.concatenate([x[1:, :], zr], axis=0)
        xu = jnp.concatenate([zr, x[:-1, :]], axis=0)
        return ib - ((ncr * xl + ncl * xr) + (ncd * xd + ncu * xu))

    mi = mi_ref[0]
    x = jax.lax.fori_loop(0, mi // 5,
                          lambda _, x: sweep(sweep(sweep(sweep(sweep(x))))),
                          x0_ref[0])
    out_ref[0] = jax.lax.fori_loop(0, mi % 5, lambda _, x: sweep(x), x)


def kernel(u, M_vals, invD_vals, b, rows, cols, maxiter):
    del rows, cols
    B = u.shape[0]
    n = u.shape[-1]
    E = n * (n - 1)
    original_shape = u.shape

    seg = M_vals.reshape(B, 4, E)
    cr = jnp.pad(seg[:, 0].reshape(B, n, n - 1), ((0, 0), (0, 0), (0, 1)))
    cl = jnp.pad(seg[:, 1].reshape(B, n, n - 1), ((0, 0), (0, 0), (1, 0)))
    cd = jnp.pad(seg[:, 2].reshape(B, n - 1, n), ((0, 0), (0, 1), (0, 0)))
    cu = jnp.pad(seg[:, 3].reshape(B, n - 1, n), ((0, 0), (1, 0), (0, 0)))

    x0 = u.reshape(B, n, n)
    invd = invD_vals.reshape(B, n, n)
    bg = b.reshape(B, n, n)
    mi = jnp.asarray(maxiter, dtype=jnp.int32).reshape(1)

    spec = pl.BlockSpec((1, n, n), lambda i, mi_: (i, 0, 0))
    out = pl.pallas_call(
        _jacobi_body,
        grid_spec=pltpu.PrefetchScalarGridSpec(
            num_scalar_prefetch=1,
            grid=(B,),
            in_specs=[spec] * 7,
            out_specs=spec,
        ),
        out_shape=jax.ShapeDtypeStruct((B, n, n), jnp.float32),
        compiler_params=pltpu.CompilerParams(
            dimension_semantics=("parallel",),
        ),
    )(mi, x0, cr, cl, cd, cu, invd, bg)

    return jax.lax.stop_gradient(out.reshape(original_shape))


# speedup vs baseline: 1.0436x; 1.0197x over previous
import jax
import jax.numpy as jnp
from jax.experimental import pallas as pl
from jax.experimental.pallas import tpu as pltpu


def _jacobi_body(mi_ref, x0_ref, cr_ref, cl_ref, cd_ref, cu_ref, invd_ref,
                 b_ref, out_ref):
    invd = invd_ref[0]
    ncr = invd * cr_ref[0]
    ncl = invd * cl_ref[0]
    ncd = invd * cd_ref[0]
    ncu = invd * cu_ref[0]
    ib = invd * b_ref[0]

    n = x0_ref.shape[1]
    zc = jnp.zeros((n, 1), dtype=jnp.float32)
    zr = jnp.zeros((1, n), dtype=jnp.float32)

    def sweep(x):
        xl = jnp.concatenate([x[:, 1:], zc], axis=1)
        xr = jnp.concatenate([zc, x[:, :-1]], axis=1)
        xd = jnp.concatenate([x[1:, :], zr], axis=0)
        xu = jnp.concatenate([zr, x[:-1, :]], axis=0)
        return ib - ((ncr * xl + ncl * xr) + (ncd * xd + ncu * xu))

    mi = mi_ref[0]

    def ten(x):
        for _ in range(10):
            x = sweep(x)
        return x

    x = jax.lax.fori_loop(0, mi // 10, lambda _, x: ten(x), x0_ref[0])
    out_ref[0] = jax.lax.fori_loop(0, mi % 10, lambda _, x: sweep(x), x)


def kernel(u, M_vals, invD_vals, b, rows, cols, maxiter):
    del rows, cols
    B = u.shape[0]
    n = u.shape[-1]
    E = n * (n - 1)
    original_shape = u.shape

    seg = M_vals.reshape(B, 4, E)
    cr = jnp.pad(seg[:, 0].reshape(B, n, n - 1), ((0, 0), (0, 0), (0, 1)))
    cl = jnp.pad(seg[:, 1].reshape(B, n, n - 1), ((0, 0), (0, 0), (1, 0)))
    cd = jnp.pad(seg[:, 2].reshape(B, n - 1, n), ((0, 0), (0, 1), (0, 0)))
    cu = jnp.pad(seg[:, 3].reshape(B, n - 1, n), ((0, 0), (1, 0), (0, 0)))

    x0 = u.reshape(B, n, n)
    invd = invD_vals.reshape(B, n, n)
    bg = b.reshape(B, n, n)
    mi = jnp.asarray(maxiter, dtype=jnp.int32).reshape(1)

    spec = pl.BlockSpec((1, n, n), lambda i, mi_: (i, 0, 0))
    out = pl.pallas_call(
        _jacobi_body,
        grid_spec=pltpu.PrefetchScalarGridSpec(
            num_scalar_prefetch=1,
            grid=(B,),
            in_specs=[spec] * 7,
            out_specs=spec,
        ),
        out_shape=jax.ShapeDtypeStruct((B, n, n), jnp.float32),
        compiler_params=pltpu.CompilerParams(
            dimension_semantics=("parallel",),
        ),
    )(mi, x0, cr, cl, cd, cu, invd, bg)

    return jax.lax.stop_gradient(out.reshape(original_shape))


# speedup vs baseline: 1.0578x; 1.0136x over previous
import jax
import jax.numpy as jnp
from jax.experimental import pallas as pl
from jax.experimental.pallas import tpu as pltpu


def _jacobi_body(mi_ref, x0_ref, cr_ref, cl_ref, cd_ref, cu_ref, invd_ref,
                 b_ref, out_ref):
    invd = invd_ref[0]
    ncr = invd * cr_ref[0]
    ncl = invd * cl_ref[0]
    ncd = invd * cd_ref[0]
    ncu = invd * cu_ref[0]
    ib = invd * b_ref[0]

    n = x0_ref.shape[1]
    zc = jnp.zeros((n, 1), dtype=jnp.float32)
    zr = jnp.zeros((1, n), dtype=jnp.float32)

    def sweep(x):
        xl = jnp.concatenate([x[:, 1:], zc], axis=1)
        xr = jnp.concatenate([zc, x[:, :-1]], axis=1)
        xd = jnp.concatenate([x[1:, :], zr], axis=0)
        xu = jnp.concatenate([zr, x[:-1, :]], axis=0)
        return ib - ((ncr * xl + ncl * xr) + (ncd * xd + ncu * xu))

    mi = mi_ref[0]

    def twenty(x):
        for _ in range(20):
            x = sweep(x)
        return x

    x = jax.lax.fori_loop(0, mi // 20, lambda _, x: twenty(x), x0_ref[0])
    out_ref[0] = jax.lax.fori_loop(0, mi % 20, lambda _, x: sweep(x), x)


def kernel(u, M_vals, invD_vals, b, rows, cols, maxiter):
    del rows, cols
    B = u.shape[0]
    n = u.shape[-1]
    E = n * (n - 1)
    original_shape = u.shape

    seg = M_vals.reshape(B, 4, E)
    cr = jnp.pad(seg[:, 0].reshape(B, n, n - 1), ((0, 0), (0, 0), (0, 1)))
    cl = jnp.pad(seg[:, 1].reshape(B, n, n - 1), ((0, 0), (0, 0), (1, 0)))
    cd = jnp.pad(seg[:, 2].reshape(B, n - 1, n), ((0, 0), (0, 1), (0, 0)))
    cu = jnp.pad(seg[:, 3].reshape(B, n - 1, n), ((0, 0), (1, 0), (0, 0)))

    x0 = u.reshape(B, n, n)
    invd = invD_vals.reshape(B, n, n)
    bg = b.reshape(B, n, n)
    mi = jnp.asarray(maxiter, dtype=jnp.int32).reshape(1)

    spec = pl.BlockSpec((1, n, n), lambda i, mi_: (i, 0, 0))
    out = pl.pallas_call(
        _jacobi_body,
        grid_spec=pltpu.PrefetchScalarGridSpec(
            num_scalar_prefetch=1,
            grid=(B,),
            in_specs=[spec] * 7,
            out_specs=spec,
        ),
        out_shape=jax.ShapeDtypeStruct((B, n, n), jnp.float32),
        compiler_params=pltpu.CompilerParams(
            dimension_semantics=("parallel",),
        ),
    )(mi, x0, cr, cl, cd, cu, invd, bg)

    return jax.lax.stop_gradient(out.reshape(original_shape))
